# Initial kernel scaffold; baseline (speedup 1.0000x reference)
import jax
import jax.numpy as jnp
from jax.experimental import pallas as pl


def kernel(x, noise, ln1_g, ln1_b, ln2_g, ln2_b, w_qkv, b_qkv, w_proj, b_proj, w_fc1, b_fc1, w_fc2, b_fc2):
    raise NotImplementedError("write your pallas kernel here")


# speedup vs baseline: 1.4557x; 1.4557x over previous
import jax
import jax.numpy as jnp
import numpy as np
from jax.experimental import pallas as pl
from jax.experimental.pallas import tpu as pltpu

_B, _N, _D = 16, 1024, 1152
_H = 16
_DH = _D // _H
_GH = _GW = 32
_SY = _SX = 2
_WS = _SY * _SX
_NW = (_GH // _SY) * (_GW // _SX)
_NS = _NW * (_WS - 1)
_NUM_PRUNE = _N // 2
_NK = _NW + (_NS - _NUM_PRUNE)
_NOISE_ALPHA = 0.1
_SIM_BETA = 1.0

_INTERPRET = False


def _prep_indices(x, noise):
    xn = x / (jnp.linalg.norm(x, axis=-1, keepdims=True) + 1e-6)
    grid = jnp.arange(_N).reshape(_GH, _GW)
    win = grid.reshape(_GH // _SY, _SY, _GW // _SX, _SX).transpose(0, 2, 1, 3).reshape(_NW, _WS)
    xw = xn[:, win]
    mean = xw.mean(2, keepdims=True)
    score = _SIM_BETA * (xw * mean).sum(-1) + _NOISE_ALPHA * noise
    dst_local = jnp.argmax(score, axis=-1)
    winb = jnp.broadcast_to(win, (_B, _NW, _WS))
    dst_idx = jnp.take_along_axis(winb, dst_local[..., None], -1)[..., 0]
    local = jnp.arange(_WS)[None, None]
    keymat = jnp.where(local != dst_local[..., None], local, _WS)
    src_local = jnp.sort(keymat, -1)[..., : _WS - 1]
    src_idx = jnp.take_along_axis(winb, src_local, -1).reshape(_B, _NS)
    a = jnp.take_along_axis(xn, src_idx[..., None], 1)
    bf = jnp.take_along_axis(xn, dst_idx[..., None], 1)
    sim = jnp.einsum('bsd,bnd->bsn', a, bf)
    best = jnp.argmax(sim, -1)
    order = jnp.argsort(-jnp.max(sim, -1), axis=-1)
    pruned_pos = order[:, :_NUM_PRUNE]
    kept_pos = order[:, _NUM_PRUNE:]
    src_kept = jnp.take_along_axis(src_idx, kept_pos, 1)
    src_pruned = jnp.take_along_axis(src_idx, pruned_pos, 1)
    pruned_best = jnp.take_along_axis(best, pruned_pos, 1)

    keep_idx = jnp.concatenate([dst_idx, src_kept], axis=1).astype(jnp.int32)
    rows = jnp.arange(_B)[:, None]
    g = jnp.zeros((_B, _N), jnp.int32)
    g = g.at[rows, dst_idx].set(jnp.broadcast_to(jnp.arange(_NW, dtype=jnp.int32), (_B, _NW)))
    g = g.at[rows, src_kept].set(
        jnp.broadcast_to(_NW + jnp.arange(_NS - _NUM_PRUNE, dtype=jnp.int32),
                         (_B, _NS - _NUM_PRUNE)))
    g = g.at[rows, src_pruned].set(pruned_best.astype(jnp.int32))
    return keep_idx, g


def _ln_rows(v, g, b):
    mu = jnp.mean(v, axis=-1, keepdims=True)
    var = jnp.mean((v - mu) ** 2, axis=-1, keepdims=True)
    return (v - mu) * jax.lax.rsqrt(var + 1e-6) * g + b


def _qkv_kernel(x_ref, keep_ref, w_ref, bias_ref, g1_ref, b1_ref,
                qkv_ref, xk_ref, y_s):
    j = pl.program_id(1)

    @pl.when(j == 0)
    def _():
        keep_row = keep_ref[0]
        iota_t = jax.lax.broadcasted_iota(jnp.int32, (_N, _NK), 0)
        oht = (iota_t == keep_row).astype(jnp.float32)
        xk = jax.lax.dot_general(oht, x_ref[0], (((0,), (0,)), ((), ())),
                                 preferred_element_type=jnp.float32)
        xk_ref[0] = xk
        y_s[...] = _ln_rows(xk, g1_ref[0], b1_ref[0])

    qkv_ref[0] = jax.lax.dot_general(
        y_s[...], w_ref[...], (((1,), (0,)), ((), ())),
        preferred_element_type=jnp.float32) + bias_ref[0]


def _attn_kernel(qkv_ref, xk_ref, wp_ref, bp_ref, x2_ref, o_s):
    qkv = qkv_ref[0]
    scale = 1.0 / np.sqrt(_DH)
    for h in range(_H):
        q = qkv[:, h * _DH:(h + 1) * _DH]
        k = qkv[:, _D + h * _DH:_D + (h + 1) * _DH]
        v = qkv[:, 2 * _D + h * _DH:2 * _D + (h + 1) * _DH]
        s = jax.lax.dot_general(q, k, (((1,), (1,)), ((), ())),
                                preferred_element_type=jnp.float32) * scale
        m = jnp.max(s, axis=1, keepdims=True)
        e = jnp.exp(s - m)
        att = e / jnp.sum(e, axis=1, keepdims=True)
        o_s[:, h * _DH:(h + 1) * _DH] = jax.lax.dot_general(
            att, v, (((1,), (0,)), ((), ())),
            preferred_element_type=jnp.float32)
    x2_ref[0] = xk_ref[0] + jax.lax.dot_general(
        o_s[...], wp_ref[...], (((1,), (0,)), ((), ())),
        preferred_element_type=jnp.float32) + bp_ref[0]


def _mlp_kernel(x2_ref, g_ref, w1_ref, b1_ref, w2_ref, b2_ref, g2_ref, bb2_ref,
                out_ref, y_s, acc_s):
    u = pl.program_id(1)

    @pl.when(u == 0)
    def _():
        y_s[...] = _ln_rows(x2_ref[0], g2_ref[0], bb2_ref[0])

    h = jax.nn.gelu(jax.lax.dot_general(
        y_s[...], w1_ref[...], (((1,), (0,)), ((), ())),
        preferred_element_type=jnp.float32) + b1_ref[0])
    contrib = jax.lax.dot_general(h, w2_ref[...], (((1,), (0,)), ((), ())),
                                  preferred_element_type=jnp.float32)

    @pl.when(u == 0)
    def _():
        acc_s[...] = contrib

    @pl.when(u != 0)
    def _():
        acc_s[...] = acc_s[...] + contrib

    @pl.when(u == 3)
    def _():
        out2 = x2_ref[0] + acc_s[...] + b2_ref[0]
        g_row = g_ref[0]
        iota_j = jax.lax.broadcasted_iota(jnp.int32, (_NK, _N), 0)
        rt = (iota_j == g_row).astype(jnp.float32)
        out_ref[0] = jax.lax.dot_general(rt, out2, (((0,), (0,)), ((), ())),
                                         preferred_element_type=jnp.float32)


def kernel(x, noise, ln1_g, ln1_b, ln2_g, ln2_b, w_qkv, b_qkv, w_proj, b_proj,
           w_fc1, b_fc1, w_fc2, b_fc2):
    keep_idx, g = _prep_indices(x, noise)
    keep3 = keep_idx.reshape(_B, 1, _NK)
    g3 = g.reshape(_B, 1, _N)
    r2 = lambda v: v.reshape(1, -1)

    qkv, xk = pl.pallas_call(
        _qkv_kernel,
        grid=(_B, 3),
        in_specs=[
            pl.BlockSpec((1, _N, _D), lambda b, j: (b, 0, 0)),
            pl.BlockSpec((1, 1, _NK), lambda b, j: (b, 0, 0)),
            pl.BlockSpec((_D, _D), lambda b, j: (0, j)),
            pl.BlockSpec((1, _D), lambda b, j: (0, j)),
            pl.BlockSpec((1, _D), lambda b, j: (0, 0)),
            pl.BlockSpec((1, _D), lambda b, j: (0, 0)),
        ],
        out_specs=[
            pl.BlockSpec((1, _NK, _D), lambda b, j: (b, 0, j)),
            pl.BlockSpec((1, _NK, _D), lambda b, j: (b, 0, 0)),
        ],
        out_shape=[
            jax.ShapeDtypeStruct((_B, _NK, 3 * _D), jnp.float32),
            jax.ShapeDtypeStruct((_B, _NK, _D), jnp.float32),
        ],
        scratch_shapes=[pltpu.VMEM((_NK, _D), jnp.float32)],
        compiler_params=pltpu.CompilerParams(
            dimension_semantics=("parallel", "arbitrary"),
            vmem_limit_bytes=52 * 1024 * 1024,
        ),
        name="sito_gather_qkv",
        interpret=_INTERPRET,
    )(x, keep3, w_qkv, r2(b_qkv), r2(ln1_g), r2(ln1_b))

    x2 = pl.pallas_call(
        _attn_kernel,
        grid=(_B,),
        in_specs=[
            pl.BlockSpec((1, _NK, 3 * _D), lambda b: (b, 0, 0)),
            pl.BlockSpec((1, _NK, _D), lambda b: (b, 0, 0)),
            pl.BlockSpec((_D, _D), lambda b: (0, 0)),
            pl.BlockSpec((1, _D), lambda b: (0, 0)),
        ],
        out_specs=pl.BlockSpec((1, _NK, _D), lambda b: (b, 0, 0)),
        out_shape=jax.ShapeDtypeStruct((_B, _NK, _D), jnp.float32),
        scratch_shapes=[pltpu.VMEM((_NK, _D), jnp.float32)],
        compiler_params=pltpu.CompilerParams(
            dimension_semantics=("parallel",),
            vmem_limit_bytes=52 * 1024 * 1024,
        ),
        name="sito_attn",
        interpret=_INTERPRET,
    )(qkv, xk, w_proj, r2(b_proj))

    out = pl.pallas_call(
        _mlp_kernel,
        grid=(_B, 4),
        in_specs=[
            pl.BlockSpec((1, _NK, _D), lambda b, u: (b, 0, 0)),
            pl.BlockSpec((1, 1, _N), lambda b, u: (b, 0, 0)),
            pl.BlockSpec((_D, _D), lambda b, u: (0, u)),
            pl.BlockSpec((1, _D), lambda b, u: (0, u)),
            pl.BlockSpec((_D, _D), lambda b, u: (u, 0)),
            pl.BlockSpec((1, _D), lambda b, u: (0, 0)),
            pl.BlockSpec((1, _D), lambda b, u: (0, 0)),
            pl.BlockSpec((1, _D), lambda b, u: (0, 0)),
        ],
        out_specs=pl.BlockSpec((1, _N, _D), lambda b, u: (b, 0, 0)),
        out_shape=jax.ShapeDtypeStruct((_B, _N, _D), jnp.float32),
        scratch_shapes=[pltpu.VMEM((_NK, _D), jnp.float32),
                        pltpu.VMEM((_NK, _D), jnp.float32)],
        compiler_params=pltpu.CompilerParams(
            dimension_semantics=("parallel", "arbitrary"),
            vmem_limit_bytes=52 * 1024 * 1024,
        ),
        name="sito_mlp_recover",
        interpret=_INTERPRET,
    )(x2, g3, w_fc1, r2(b_fc1), w_fc2, r2(b_fc2), r2(ln2_g), r2(ln2_b))

    return out


# speedup vs baseline: 1.4822x; 1.0182x over previous
import jax
import jax.numpy as jnp
import numpy as np
from jax.experimental import pallas as pl
from jax.experimental.pallas import tpu as pltpu

_B, _N, _D = 16, 1024, 1152
_H = 16
_DH = _D // _H
_GH = _GW = 32
_SY = _SX = 2
_WS = _SY * _SX
_NW = (_GH // _SY) * (_GW // _SX)
_NS = _NW * (_WS - 1)
_NUM_PRUNE = _N // 2
_NK = _NW + (_NS - _NUM_PRUNE)
_NOISE_ALPHA = 0.1
_SIM_BETA = 1.0

_INTERPRET = False


def _prep_indices(x, noise):
    xn = x / (jnp.linalg.norm(x, axis=-1, keepdims=True) + 1e-6)
    grid = jnp.arange(_N).reshape(_GH, _GW)
    win = grid.reshape(_GH // _SY, _SY, _GW // _SX, _SX).transpose(0, 2, 1, 3).reshape(_NW, _WS)
    xw = xn[:, win]
    mean = xw.mean(2, keepdims=True)
    score = _SIM_BETA * (xw * mean).sum(-1) + _NOISE_ALPHA * noise
    dst_local = jnp.argmax(score, axis=-1)
    winb = jnp.broadcast_to(win, (_B, _NW, _WS))
    dst_idx = jnp.take_along_axis(winb, dst_local[..., None], -1)[..., 0]
    local = jnp.arange(_WS)[None, None]
    keymat = jnp.where(local != dst_local[..., None], local, _WS)
    src_local = jnp.sort(keymat, -1)[..., : _WS - 1]
    src_idx = jnp.take_along_axis(winb, src_local, -1).reshape(_B, _NS)
    a = jnp.take_along_axis(xn, src_idx[..., None], 1)
    bf = jnp.take_along_axis(xn, dst_idx[..., None], 1)
    sim = jnp.einsum('bsd,bnd->bsn', a, bf)
    best = jnp.argmax(sim, -1)
    order = jnp.argsort(-jnp.max(sim, -1), axis=-1)
    pruned_pos = order[:, :_NUM_PRUNE]
    kept_pos = order[:, _NUM_PRUNE:]
    src_kept = jnp.take_along_axis(src_idx, kept_pos, 1)
    src_pruned = jnp.take_along_axis(src_idx, pruned_pos, 1)
    pruned_best = jnp.take_along_axis(best, pruned_pos, 1)

    keep_idx = jnp.concatenate([dst_idx, src_kept], axis=1).astype(jnp.int32)
    rows = jnp.arange(_B)[:, None]
    g = jnp.zeros((_B, _N), jnp.int32)
    g = g.at[rows, dst_idx].set(jnp.broadcast_to(jnp.arange(_NW, dtype=jnp.int32), (_B, _NW)))
    g = g.at[rows, src_kept].set(
        jnp.broadcast_to(_NW + jnp.arange(_NS - _NUM_PRUNE, dtype=jnp.int32),
                         (_B, _NS - _NUM_PRUNE)))
    g = g.at[rows, src_pruned].set(pruned_best.astype(jnp.int32))
    return keep_idx, g


def _ln_rows(v, g, b):
    mu = jnp.mean(v, axis=-1, keepdims=True)
    var = jnp.mean((v - mu) ** 2, axis=-1, keepdims=True)
    return (v - mu) * jax.lax.rsqrt(var + 1e-6) * g + b


def _qkv_kernel(x_ref, keep_ref, w_ref, bias_ref, g1_ref, b1_ref,
                qkv_ref, xk_ref, y_s):
    j = pl.program_id(1)

    @pl.when(j == 0)
    def _():
        keep_row = keep_ref[0]
        iota_t = jax.lax.broadcasted_iota(jnp.int32, (_N, _NK), 0)
        oht = (iota_t == keep_row).astype(jnp.float32)
        xk = jax.lax.dot_general(oht, x_ref[0], (((0,), (0,)), ((), ())),
                                 preferred_element_type=jnp.float32)
        xk_ref[0] = xk
        y_s[...] = _ln_rows(xk, g1_ref[0], b1_ref[0]).astype(jnp.bfloat16)

    qkv_ref[0] = (jax.lax.dot_general(
        y_s[...], w_ref[...], (((1,), (0,)), ((), ())),
        preferred_element_type=jnp.float32) + bias_ref[0]).astype(jnp.bfloat16)


def _attn_kernel(qkv_ref, xk_ref, wp_ref, bp_ref, x2_ref, o_s):
    qkv = qkv_ref[0]
    scale = 1.0 / np.sqrt(_DH)
    for h in range(_H):
        q = qkv[:, h * _DH:(h + 1) * _DH]
        k = qkv[:, _D + h * _DH:_D + (h + 1) * _DH]
        v = qkv[:, 2 * _D + h * _DH:2 * _D + (h + 1) * _DH]
        s = jax.lax.dot_general(q, k, (((1,), (1,)), ((), ())),
                                preferred_element_type=jnp.float32) * scale
        m = jnp.max(s, axis=1, keepdims=True)
        e = jnp.exp(s - m)
        att = (e / jnp.sum(e, axis=1, keepdims=True)).astype(jnp.bfloat16)
        o_s[:, h * _DH:(h + 1) * _DH] = jax.lax.dot_general(
            att, v, (((1,), (0,)), ((), ())),
            preferred_element_type=jnp.float32).astype(jnp.bfloat16)
    x2_ref[0] = xk_ref[0] + jax.lax.dot_general(
        o_s[...], wp_ref[...], (((1,), (0,)), ((), ())),
        preferred_element_type=jnp.float32) + bp_ref[0]


def _mlp_kernel(x2_ref, g_ref, w1_ref, b1_ref, w2_ref, b2_ref, g2_ref, bb2_ref,
                out_ref, y_s, acc_s):
    u = pl.program_id(1)

    @pl.when(u == 0)
    def _():
        y_s[...] = _ln_rows(x2_ref[0], g2_ref[0], bb2_ref[0]).astype(jnp.bfloat16)

    h = jax.nn.gelu(jax.lax.dot_general(
        y_s[...], w1_ref[...], (((1,), (0,)), ((), ())),
        preferred_element_type=jnp.float32) + b1_ref[0]).astype(jnp.bfloat16)
    contrib = jax.lax.dot_general(h, w2_ref[...], (((1,), (0,)), ((), ())),
                                  preferred_element_type=jnp.float32)

    @pl.when(u == 0)
    def _():
        acc_s[...] = contrib

    @pl.when(u != 0)
    def _():
        acc_s[...] = acc_s[...] + contrib

    @pl.when(u == 3)
    def _():
        out2 = x2_ref[0] + acc_s[...] + b2_ref[0]
        g_row = g_ref[0]
        iota_j = jax.lax.broadcasted_iota(jnp.int32, (_NK, _N), 0)
        rt = (iota_j == g_row).astype(jnp.float32)
        out_ref[0] = jax.lax.dot_general(rt, out2, (((0,), (0,)), ((), ())),
                                         preferred_element_type=jnp.float32)


def kernel(x, noise, ln1_g, ln1_b, ln2_g, ln2_b, w_qkv, b_qkv, w_proj, b_proj,
           w_fc1, b_fc1, w_fc2, b_fc2):
    keep_idx, g = _prep_indices(x, noise)
    keep3 = keep_idx.reshape(_B, 1, _NK)
    g3 = g.reshape(_B, 1, _N)
    r2 = lambda v: v.reshape(1, -1)
    bf = lambda v: v.astype(jnp.bfloat16)

    qkv, xk = pl.pallas_call(
        _qkv_kernel,
        grid=(_B, 3),
        in_specs=[
            pl.BlockSpec((1, _N, _D), lambda b, j: (b, 0, 0)),
            pl.BlockSpec((1, 1, _NK), lambda b, j: (b, 0, 0)),
            pl.BlockSpec((_D, _D), lambda b, j: (0, j)),
            pl.BlockSpec((1, _D), lambda b, j: (0, j)),
            pl.BlockSpec((1, _D), lambda b, j: (0, 0)),
            pl.BlockSpec((1, _D), lambda b, j: (0, 0)),
        ],
        out_specs=[
            pl.BlockSpec((1, _NK, _D), lambda b, j: (b, 0, j)),
            pl.BlockSpec((1, _NK, _D), lambda b, j: (b, 0, 0)),
        ],
        out_shape=[
            jax.ShapeDtypeStruct((_B, _NK, 3 * _D), jnp.bfloat16),
            jax.ShapeDtypeStruct((_B, _NK, _D), jnp.float32),
        ],
        scratch_shapes=[pltpu.VMEM((_NK, _D), jnp.bfloat16)],
        compiler_params=pltpu.CompilerParams(
            dimension_semantics=("parallel", "arbitrary"),
            vmem_limit_bytes=52 * 1024 * 1024,
        ),
        name="sito_gather_qkv",
        interpret=_INTERPRET,
    )(x, keep3, bf(w_qkv), r2(b_qkv), r2(ln1_g), r2(ln1_b))

    x2 = pl.pallas_call(
        _attn_kernel,
        grid=(_B,),
        in_specs=[
            pl.BlockSpec((1, _NK, 3 * _D), lambda b: (b, 0, 0)),
            pl.BlockSpec((1, _NK, _D), lambda b: (b, 0, 0)),
            pl.BlockSpec((_D, _D), lambda b: (0, 0)),
            pl.BlockSpec((1, _D), lambda b: (0, 0)),
        ],
        out_specs=pl.BlockSpec((1, _NK, _D), lambda b: (b, 0, 0)),
        out_shape=jax.ShapeDtypeStruct((_B, _NK, _D), jnp.float32),
        scratch_shapes=[pltpu.VMEM((_NK, _D), jnp.bfloat16)],
        compiler_params=pltpu.CompilerParams(
            dimension_semantics=("parallel",),
            vmem_limit_bytes=52 * 1024 * 1024,
        ),
        name="sito_attn",
        interpret=_INTERPRET,
    )(qkv, xk, bf(w_proj), r2(b_proj))

    out = pl.pallas_call(
        _mlp_kernel,
        grid=(_B, 4),
        in_specs=[
            pl.BlockSpec((1, _NK, _D), lambda b, u: (b, 0, 0)),
            pl.BlockSpec((1, 1, _N), lambda b, u: (b, 0, 0)),
            pl.BlockSpec((_D, _D), lambda b, u: (0, u)),
            pl.BlockSpec((1, _D), lambda b, u: (0, u)),
            pl.BlockSpec((_D, _D), lambda b, u: (u, 0)),
            pl.BlockSpec((1, _D), lambda b, u: (0, 0)),
            pl.BlockSpec((1, _D), lambda b, u: (0, 0)),
            pl.BlockSpec((1, _D), lambda b, u: (0, 0)),
        ],
        out_specs=pl.BlockSpec((1, _N, _D), lambda b, u: (b, 0, 0)),
        out_shape=jax.ShapeDtypeStruct((_B, _N, _D), jnp.float32),
        scratch_shapes=[pltpu.VMEM((_NK, _D), jnp.bfloat16),
                        pltpu.VMEM((_NK, _D), jnp.float32)],
        compiler_params=pltpu.CompilerParams(
            dimension_semantics=("parallel", "arbitrary"),
            vmem_limit_bytes=52 * 1024 * 1024,
        ),
        name="sito_mlp_recover",
        interpret=_INTERPRET,
    )(x2, g3, bf(w_fc1), r2(b_fc1), bf(w_fc2), r2(b_fc2), r2(ln2_g), r2(ln2_b))

    return out


# speedup vs baseline: 1.6869x; 1.1381x over previous
import jax
import jax.numpy as jnp
import numpy as np
from jax.experimental import pallas as pl
from jax.experimental.pallas import tpu as pltpu

_B, _N, _D = 16, 1024, 1152
_H = 16
_DH = _D // _H
_GH = _GW = 32
_SY = _SX = 2
_WS = _SY * _SX
_NW = (_GH // _SY) * (_GW // _SX)
_NS = _NW * (_WS - 1)
_NUM_PRUNE = _N // 2
_NK = _NW + (_NS - _NUM_PRUNE)
_DF = 4 * _D
_NOISE_ALPHA = 0.1
_SIM_BETA = 1.0

_INTERPRET = False


def _prep_indices(x, noise):
    xn = x / (jnp.linalg.norm(x, axis=-1, keepdims=True) + 1e-6)
    grid = jnp.arange(_N).reshape(_GH, _GW)
    win = grid.reshape(_GH // _SY, _SY, _GW // _SX, _SX).transpose(0, 2, 1, 3).reshape(_NW, _WS)
    xw = xn[:, win]
    mean = xw.mean(2, keepdims=True)
    score = _SIM_BETA * (xw * mean).sum(-1) + _NOISE_ALPHA * noise
    dst_local = jnp.argmax(score, axis=-1)
    winb = jnp.broadcast_to(win, (_B, _NW, _WS))
    dst_idx = jnp.take_along_axis(winb, dst_local[..., None], -1)[..., 0]
    local = jnp.arange(_WS)[None, None]
    keymat = jnp.where(local != dst_local[..., None], local, _WS)
    src_local = jnp.sort(keymat, -1)[..., : _WS - 1]
    src_idx = jnp.take_along_axis(winb, src_local, -1).reshape(_B, _NS)
    a = jnp.take_along_axis(xn, src_idx[..., None], 1)
    bf = jnp.take_along_axis(xn, dst_idx[..., None], 1)
    sim = jnp.einsum('bsd,bnd->bsn', a, bf)
    best = jnp.argmax(sim, -1)
    order = jnp.argsort(-jnp.max(sim, -1), axis=-1)
    pruned_pos = order[:, :_NUM_PRUNE]
    kept_pos = order[:, _NUM_PRUNE:]
    src_kept = jnp.take_along_axis(src_idx, kept_pos, 1)
    src_pruned = jnp.take_along_axis(src_idx, pruned_pos, 1)
    pruned_best = jnp.take_along_axis(best, pruned_pos, 1)

    keep_idx = jnp.concatenate([dst_idx, src_kept], axis=1).astype(jnp.int32)
    rows = jnp.arange(_B)[:, None]
    g = jnp.zeros((_B, _N), jnp.int32)
    g = g.at[rows, dst_idx].set(jnp.broadcast_to(jnp.arange(_NW, dtype=jnp.int32), (_B, _NW)))
    g = g.at[rows, src_kept].set(
        jnp.broadcast_to(_NW + jnp.arange(_NS - _NUM_PRUNE, dtype=jnp.int32),
                         (_B, _NS - _NUM_PRUNE)))
    g = g.at[rows, src_pruned].set(pruned_best.astype(jnp.int32))
    return keep_idx, g


def _ln_rows(v, g, b):
    mu = jnp.mean(v, axis=-1, keepdims=True)
    var = jnp.mean((v - mu) ** 2, axis=-1, keepdims=True)
    return (v - mu) * jax.lax.rsqrt(var + 1e-6) * g + b


def _dot(a, b, dims):
    return jax.lax.dot_general(a, b, (dims, ((), ())),
                               preferred_element_type=jnp.float32)


def _attn_body(x_ref, keep_ref, wq_hbm, bq_ref, g1_ref, b1_ref, wp_hbm, bp_ref,
               x2_ref, wq_s, wp_s, qkv_s, xk_s, o_s, sem1, sem2):
    b = pl.program_id(0)

    @pl.when(b == 0)
    def _():
        c1 = pltpu.make_async_copy(wq_hbm, wq_s, sem1)
        c2 = pltpu.make_async_copy(wp_hbm, wp_s, sem2)
        c1.start()
        c2.start()
        c1.wait()
        c2.wait()

    keep_row = keep_ref[0]
    iota_t = jax.lax.broadcasted_iota(jnp.int32, (_N, _NK), 0)
    oht = (iota_t == keep_row).astype(jnp.float32)
    xk_s[...] = _dot(oht, x_ref[0], ((0,), (0,)))
    y = _ln_rows(xk_s[...], g1_ref[0], b1_ref[0]).astype(jnp.bfloat16)
    qkv_s[...] = (_dot(y, wq_s[...], ((1,), (0,))) + bq_ref[0]).astype(jnp.bfloat16)

    scale = 1.0 / np.sqrt(_DH)
    qkv = qkv_s[...]
    for h in range(_H):
        q = qkv[:, h * _DH:(h + 1) * _DH]
        k = qkv[:, _D + h * _DH:_D + (h + 1) * _DH]
        v = qkv[:, 2 * _D + h * _DH:2 * _D + (h + 1) * _DH]
        s = _dot(q, k, ((1,), (1,))) * scale
        m = jnp.max(s, axis=1, keepdims=True)
        e = jnp.exp(s - m)
        att = (e / jnp.sum(e, axis=1, keepdims=True)).astype(jnp.bfloat16)
        o_s[:, h * _DH:(h + 1) * _DH] = _dot(att, v, ((1,), (0,))).astype(jnp.bfloat16)

    x2_ref[0] = xk_s[...] + _dot(o_s[...], wp_s[...], ((1,), (0,))) + bp_ref[0]


def _mlp_body(x2_ref, g_ref, w1_hbm, b1_ref, w2_hbm, b2_ref, g2_ref, bb2_ref,
              out_ref, w1_s, w2_s, sem1, sem2):
    b = pl.program_id(0)

    @pl.when(b == 0)
    def _():
        c1 = pltpu.make_async_copy(w1_hbm, w1_s, sem1)
        c2 = pltpu.make_async_copy(w2_hbm, w2_s, sem2)
        c1.start()
        c2.start()
        c1.wait()
        c2.wait()

    x2 = x2_ref[0]
    y2 = _ln_rows(x2, g2_ref[0], bb2_ref[0]).astype(jnp.bfloat16)
    hh = _DF // 2
    acc = None
    for u in range(2):
        h = jax.nn.gelu(_dot(y2, w1_s[:, u * hh:(u + 1) * hh], ((1,), (0,)))
                        + b1_ref[0, u * hh:(u + 1) * hh]).astype(jnp.bfloat16)
        c = _dot(h, w2_s[u * hh:(u + 1) * hh, :], ((1,), (0,)))
        acc = c if acc is None else acc + c
    out2 = x2 + acc + b2_ref[0]

    g_row = g_ref[0]
    iota_j = jax.lax.broadcasted_iota(jnp.int32, (_NK, _N), 0)
    rt = (iota_j == g_row).astype(jnp.float32)
    out_ref[0] = _dot(rt, out2, ((0,), (0,)))


def kernel(x, noise, ln1_g, ln1_b, ln2_g, ln2_b, w_qkv, b_qkv, w_proj, b_proj,
           w_fc1, b_fc1, w_fc2, b_fc2):
    keep_idx, g = _prep_indices(x, noise)
    keep3 = keep_idx.reshape(_B, 1, _NK)
    g3 = g.reshape(_B, 1, _N)
    r2 = lambda v: v.reshape(1, -1)
    bf = lambda v: v.astype(jnp.bfloat16)

    x2 = pl.pallas_call(
        _attn_body,
        grid=(_B,),
        in_specs=[
            pl.BlockSpec((1, _N, _D), lambda b: (b, 0, 0)),
            pl.BlockSpec((1, 1, _NK), lambda b: (b, 0, 0)),
            pl.BlockSpec(memory_space=pl.ANY),
            pl.BlockSpec((1, 3 * _D), lambda b: (0, 0)),
            pl.BlockSpec((1, _D), lambda b: (0, 0)),
            pl.BlockSpec((1, _D), lambda b: (0, 0)),
            pl.BlockSpec(memory_space=pl.ANY),
            pl.BlockSpec((1, _D), lambda b: (0, 0)),
        ],
        out_specs=pl.BlockSpec((1, _NK, _D), lambda b: (b, 0, 0)),
        out_shape=jax.ShapeDtypeStruct((_B, _NK, _D), jnp.float32),
        scratch_shapes=[
            pltpu.VMEM((_D, 3 * _D), jnp.bfloat16),
            pltpu.VMEM((_D, _D), jnp.bfloat16),
            pltpu.VMEM((_NK, 3 * _D), jnp.bfloat16),
            pltpu.VMEM((_NK, _D), jnp.float32),
            pltpu.VMEM((_NK, _D), jnp.bfloat16),
            pltpu.SemaphoreType.DMA,
            pltpu.SemaphoreType.DMA,
        ],
        compiler_params=pltpu.CompilerParams(
            dimension_semantics=("arbitrary",),
            vmem_limit_bytes=56 * 1024 * 1024,
        ),
        name="sito_gather_qkv_attn",
        interpret=_INTERPRET,
    )(x, keep3, bf(w_qkv), r2(b_qkv), r2(ln1_g), r2(ln1_b), bf(w_proj), r2(b_proj))

    out = pl.pallas_call(
        _mlp_body,
        grid=(_B,),
        in_specs=[
            pl.BlockSpec((1, _NK, _D), lambda b: (b, 0, 0)),
            pl.BlockSpec((1, 1, _N), lambda b: (b, 0, 0)),
            pl.BlockSpec(memory_space=pl.ANY),
            pl.BlockSpec((1, _DF), lambda b: (0, 0)),
            pl.BlockSpec(memory_space=pl.ANY),
            pl.BlockSpec((1, _D), lambda b: (0, 0)),
            pl.BlockSpec((1, _D), lambda b: (0, 0)),
            pl.BlockSpec((1, _D), lambda b: (0, 0)),
        ],
        out_specs=pl.BlockSpec((1, _N, _D), lambda b: (b, 0, 0)),
        out_shape=jax.ShapeDtypeStruct((_B, _N, _D), jnp.float32),
        scratch_shapes=[
            pltpu.VMEM((_D, _DF), jnp.bfloat16),
            pltpu.VMEM((_DF, _D), jnp.bfloat16),
            pltpu.SemaphoreType.DMA,
            pltpu.SemaphoreType.DMA,
        ],
        compiler_params=pltpu.CompilerParams(
            dimension_semantics=("arbitrary",),
            vmem_limit_bytes=56 * 1024 * 1024,
        ),
        name="sito_mlp_recover",
        interpret=_INTERPRET,
    )(x2, g3, bf(w_fc1), r2(b_fc1), bf(w_fc2), r2(b_fc2), r2(ln2_g), r2(ln2_b))

    return out


# speedup vs baseline: 1.7443x; 1.0340x over previous
import jax
import jax.numpy as jnp
import numpy as np
from jax.experimental import pallas as pl
from jax.experimental.pallas import tpu as pltpu

_B, _N, _D = 16, 1024, 1152
_H = 16
_DH = _D // _H
_GH = _GW = 32
_SY = _SX = 2
_WS = _SY * _SX
_NW = (_GH // _SY) * (_GW // _SX)
_NS = _NW * (_WS - 1)
_NUM_PRUNE = _N // 2
_NK = _NW + (_NS - _NUM_PRUNE)
_DF = 4 * _D
_NOISE_ALPHA = 0.1
_SIM_BETA = 1.0

_INTERPRET = False


def _prep_indices(x, noise):
    xn = x / (jnp.linalg.norm(x, axis=-1, keepdims=True) + 1e-6)
    grid = jnp.arange(_N).reshape(_GH, _GW)
    win = grid.reshape(_GH // _SY, _SY, _GW // _SX, _SX).transpose(0, 2, 1, 3).reshape(_NW, _WS)
    xw = xn[:, win]
    mean = xw.mean(2, keepdims=True)
    score = _SIM_BETA * (xw * mean).sum(-1) + _NOISE_ALPHA * noise
    dst_local = jnp.argmax(score, axis=-1)
    winb = jnp.broadcast_to(win, (_B, _NW, _WS))
    dst_idx = jnp.take_along_axis(winb, dst_local[..., None], -1)[..., 0]
    local = jnp.arange(_WS)[None, None]
    keymat = jnp.where(local != dst_local[..., None], local, _WS)
    src_local = jnp.sort(keymat, -1)[..., : _WS - 1]
    src_idx = jnp.take_along_axis(winb, src_local, -1).reshape(_B, _NS)
    a = jnp.take_along_axis(xn, src_idx[..., None], 1)
    bf = jnp.take_along_axis(xn, dst_idx[..., None], 1)
    sim = jnp.einsum('bsd,bnd->bsn', a, bf)
    best = jnp.argmax(sim, -1)
    order = jnp.argsort(-jnp.max(sim, -1), axis=-1)
    pruned_pos = order[:, :_NUM_PRUNE]
    kept_pos = order[:, _NUM_PRUNE:]
    src_kept = jnp.take_along_axis(src_idx, kept_pos, 1)
    src_pruned = jnp.take_along_axis(src_idx, pruned_pos, 1)
    pruned_best = jnp.take_along_axis(best, pruned_pos, 1)

    keep_idx = jnp.concatenate([dst_idx, src_kept], axis=1).astype(jnp.int32)
    rows = jnp.arange(_B)[:, None]
    g = jnp.zeros((_B, _N), jnp.int32)
    g = g.at[rows, dst_idx].set(jnp.broadcast_to(jnp.arange(_NW, dtype=jnp.int32), (_B, _NW)))
    g = g.at[rows, src_kept].set(
        jnp.broadcast_to(_NW + jnp.arange(_NS - _NUM_PRUNE, dtype=jnp.int32),
                         (_B, _NS - _NUM_PRUNE)))
    g = g.at[rows, src_pruned].set(pruned_best.astype(jnp.int32))
    return keep_idx, g


def _ln_rows(v, g, b):
    mu = jnp.mean(v, axis=-1, keepdims=True)
    var = jnp.mean((v - mu) ** 2, axis=-1, keepdims=True)
    return (v - mu) * jax.lax.rsqrt(var + 1e-6) * g + b


def _dot(a, b, dims):
    return jax.lax.dot_general(a, b, (dims, ((), ())),
                               preferred_element_type=jnp.float32)


def _attn_body(x_ref, keep_ref, wq_hbm, bq_ref, g1_ref, b1_ref, wp_hbm, bp_ref,
               x2_ref, wq_s, wp_s, qkv_s, xk_s, o_s, sem1, sem2):
    b = pl.program_id(0)

    @pl.when(b == 0)
    def _():
        c1 = pltpu.make_async_copy(wq_hbm, wq_s, sem1)
        c2 = pltpu.make_async_copy(wp_hbm, wp_s, sem2)
        c1.start()
        c2.start()
        c1.wait()
        c2.wait()

    keep_row = keep_ref[0]
    iota_t = jax.lax.broadcasted_iota(jnp.int32, (_N, _NK), 0)
    oht = (iota_t == keep_row).astype(jnp.bfloat16)
    xk_s[...] = _dot(oht, x_ref[0], ((0,), (0,))).astype(jnp.bfloat16)
    y = _ln_rows(xk_s[...].astype(jnp.float32), g1_ref[0], b1_ref[0]).astype(jnp.bfloat16)
    qkv_s[...] = (_dot(y, wq_s[...], ((1,), (0,))) + bq_ref[0]).astype(jnp.bfloat16)

    scale = 1.0 / np.sqrt(_DH)
    qkv = qkv_s[...]
    for h in range(_H):
        q = qkv[:, h * _DH:(h + 1) * _DH]
        k = qkv[:, _D + h * _DH:_D + (h + 1) * _DH]
        v = qkv[:, 2 * _D + h * _DH:2 * _D + (h + 1) * _DH]
        e = jnp.exp(_dot(q, k, ((1,), (1,))) * scale)
        att = (e / jnp.sum(e, axis=1, keepdims=True)).astype(jnp.bfloat16)
        o_s[:, h * _DH:(h + 1) * _DH] = _dot(att, v, ((1,), (0,))).astype(jnp.bfloat16)

    x2_ref[0] = (xk_s[...].astype(jnp.float32)
                 + _dot(o_s[...], wp_s[...], ((1,), (0,))) + bp_ref[0]
                 ).astype(jnp.bfloat16)


def _mlp_body(x2_ref, g_ref, w1_hbm, b1_ref, w2_hbm, b2_ref, g2_ref, bb2_ref,
              out_ref, w1_s, w2_s, sem1, sem2):
    b = pl.program_id(0)

    @pl.when(b == 0)
    def _():
        c1 = pltpu.make_async_copy(w1_hbm, w1_s, sem1)
        c2 = pltpu.make_async_copy(w2_hbm, w2_s, sem2)
        c1.start()
        c2.start()
        c1.wait()
        c2.wait()

    x2 = x2_ref[0].astype(jnp.float32)
    y2 = _ln_rows(x2, g2_ref[0], bb2_ref[0]).astype(jnp.bfloat16)
    hh = _DF // 2
    acc = None
    for u in range(2):
        h = jax.nn.gelu(_dot(y2, w1_s[:, u * hh:(u + 1) * hh], ((1,), (0,)))
                        + b1_ref[0, u * hh:(u + 1) * hh]).astype(jnp.bfloat16)
        c = _dot(h, w2_s[u * hh:(u + 1) * hh, :], ((1,), (0,)))
        acc = c if acc is None else acc + c
    out2 = (x2 + acc + b2_ref[0]).astype(jnp.bfloat16)

    g_row = g_ref[0]
    iota_j = jax.lax.broadcasted_iota(jnp.int32, (_NK, _N), 0)
    rt = (iota_j == g_row).astype(jnp.bfloat16)
    out_ref[0] = _dot(rt, out2, ((0,), (0,)))


def kernel(x, noise, ln1_g, ln1_b, ln2_g, ln2_b, w_qkv, b_qkv, w_proj, b_proj,
           w_fc1, b_fc1, w_fc2, b_fc2):
    keep_idx, g = _prep_indices(x, noise)
    keep3 = keep_idx.reshape(_B, 1, _NK)
    g3 = g.reshape(_B, 1, _N)
    r2 = lambda v: v.reshape(1, -1)
    bf = lambda v: v.astype(jnp.bfloat16)

    x2 = pl.pallas_call(
        _attn_body,
        grid=(_B,),
        in_specs=[
            pl.BlockSpec((1, _N, _D), lambda b: (b, 0, 0)),
            pl.BlockSpec((1, 1, _NK), lambda b: (b, 0, 0)),
            pl.BlockSpec(memory_space=pl.ANY),
            pl.BlockSpec((1, 3 * _D), lambda b: (0, 0)),
            pl.BlockSpec((1, _D), lambda b: (0, 0)),
            pl.BlockSpec((1, _D), lambda b: (0, 0)),
            pl.BlockSpec(memory_space=pl.ANY),
            pl.BlockSpec((1, _D), lambda b: (0, 0)),
        ],
        out_specs=pl.BlockSpec((1, _NK, _D), lambda b: (b, 0, 0)),
        out_shape=jax.ShapeDtypeStruct((_B, _NK, _D), jnp.bfloat16),
        scratch_shapes=[
            pltpu.VMEM((_D, 3 * _D), jnp.bfloat16),
            pltpu.VMEM((_D, _D), jnp.bfloat16),
            pltpu.VMEM((_NK, 3 * _D), jnp.bfloat16),
            pltpu.VMEM((_NK, _D), jnp.bfloat16),
            pltpu.VMEM((_NK, _D), jnp.bfloat16),
            pltpu.SemaphoreType.DMA,
            pltpu.SemaphoreType.DMA,
        ],
        compiler_params=pltpu.CompilerParams(
            dimension_semantics=("arbitrary",),
            vmem_limit_bytes=56 * 1024 * 1024,
        ),
        name="sito_gather_qkv_attn",
        interpret=_INTERPRET,
    )(bf(x), keep3, bf(w_qkv), r2(b_qkv), r2(ln1_g), r2(ln1_b), bf(w_proj), r2(b_proj))

    out = pl.pallas_call(
        _mlp_body,
        grid=(_B,),
        in_specs=[
            pl.BlockSpec((1, _NK, _D), lambda b: (b, 0, 0)),
            pl.BlockSpec((1, 1, _N), lambda b: (b, 0, 0)),
            pl.BlockSpec(memory_space=pl.ANY),
            pl.BlockSpec((1, _DF), lambda b: (0, 0)),
            pl.BlockSpec(memory_space=pl.ANY),
            pl.BlockSpec((1, _D), lambda b: (0, 0)),
            pl.BlockSpec((1, _D), lambda b: (0, 0)),
            pl.BlockSpec((1, _D), lambda b: (0, 0)),
        ],
        out_specs=pl.BlockSpec((1, _N, _D), lambda b: (b, 0, 0)),
        out_shape=jax.ShapeDtypeStruct((_B, _N, _D), jnp.float32),
        scratch_shapes=[
            pltpu.VMEM((_D, _DF), jnp.bfloat16),
            pltpu.VMEM((_DF, _D), jnp.bfloat16),
            pltpu.SemaphoreType.DMA,
            pltpu.SemaphoreType.DMA,
        ],
        compiler_params=pltpu.CompilerParams(
            dimension_semantics=("arbitrary",),
            vmem_limit_bytes=56 * 1024 * 1024,
        ),
        name="sito_mlp_recover",
        interpret=_INTERPRET,
    )(x2, g3, bf(w_fc1), r2(b_fc1), bf(w_fc2), r2(b_fc2), r2(ln2_g), r2(ln2_b))

    return out


# speedup vs baseline: 1.7551x; 1.0062x over previous
import jax
import jax.numpy as jnp
import numpy as np
from jax.experimental import pallas as pl
from jax.experimental.pallas import tpu as pltpu

_B, _N, _D = 16, 1024, 1152
_H = 16
_DH = _D // _H
_GH = _GW = 32
_SY = _SX = 2
_WS = _SY * _SX
_NW = (_GH // _SY) * (_GW // _SX)
_NS = _NW * (_WS - 1)
_NUM_PRUNE = _N // 2
_NK = _NW + (_NS - _NUM_PRUNE)
_DF = 4 * _D
_NOISE_ALPHA = 0.1
_SIM_BETA = 1.0

_INTERPRET = False


def _prep_indices(x, noise):
    xn = x / (jnp.linalg.norm(x, axis=-1, keepdims=True) + 1e-6)
    grid = jnp.arange(_N).reshape(_GH, _GW)
    win = grid.reshape(_GH // _SY, _SY, _GW // _SX, _SX).transpose(0, 2, 1, 3).reshape(_NW, _WS)
    xw = xn[:, win]
    mean = xw.mean(2, keepdims=True)
    score = _SIM_BETA * (xw * mean).sum(-1) + _NOISE_ALPHA * noise
    dst_local = jnp.argmax(score, axis=-1)
    winb = jnp.broadcast_to(win, (_B, _NW, _WS))
    dst_idx = jnp.take_along_axis(winb, dst_local[..., None], -1)[..., 0]
    local = jnp.arange(_WS)[None, None]
    keymat = jnp.where(local != dst_local[..., None], local, _WS)
    src_local = jnp.sort(keymat, -1)[..., : _WS - 1]
    src_idx = jnp.take_along_axis(winb, src_local, -1).reshape(_B, _NS)
    a = jnp.take_along_axis(xn, src_idx[..., None], 1)
    bf = jnp.take_along_axis(xn, dst_idx[..., None], 1)
    sim = jnp.einsum('bsd,bnd->bsn', a, bf)
    best = jnp.argmax(sim, -1)
    order = jnp.argsort(-jnp.max(sim, -1), axis=-1)
    src_ord = jnp.take_along_axis(src_idx, order, 1)
    best_ord = jnp.take_along_axis(best, order, 1)
    src_pruned = src_ord[:, :_NUM_PRUNE]
    src_kept = src_ord[:, _NUM_PRUNE:]
    pruned_best = best_ord[:, :_NUM_PRUNE]

    keep_idx = jnp.concatenate([dst_idx, src_kept], axis=1).astype(jnp.int32)
    rows = jnp.arange(_B)[:, None]
    scat_idx = jnp.concatenate([keep_idx, src_pruned], axis=1)
    scat_val = jnp.concatenate(
        [jnp.broadcast_to(jnp.arange(_NK, dtype=jnp.int32), (_B, _NK)),
         pruned_best.astype(jnp.int32)], axis=1)
    g = jnp.zeros((_B, _N), jnp.int32).at[rows, scat_idx].set(scat_val)
    return keep_idx, g


def _ln_rows(v, g, b):
    mu = jnp.mean(v, axis=-1, keepdims=True)
    var = jnp.mean((v - mu) ** 2, axis=-1, keepdims=True)
    return (v - mu) * jax.lax.rsqrt(var + 1e-6) * g + b


def _dot(a, b, dims):
    return jax.lax.dot_general(a, b, (dims, ((), ())),
                               preferred_element_type=jnp.float32)


def _attn_body(x_ref, keep_ref, wq_hbm, bq_ref, g1_ref, b1_ref, wp_hbm, bp_ref,
               x2_ref, wq_s, wp_s, qkv_s, xk_s, o_s, sem1, sem2):
    b = pl.program_id(0)

    @pl.when(b == 0)
    def _():
        c1 = pltpu.make_async_copy(wq_hbm, wq_s, sem1)
        c2 = pltpu.make_async_copy(wp_hbm, wp_s, sem2)
        c1.start()
        c2.start()
        c1.wait()
        c2.wait()

    keep_row = keep_ref[0]
    iota_t = jax.lax.broadcasted_iota(jnp.int32, (_N, _NK), 0)
    oht = (iota_t == keep_row).astype(jnp.bfloat16)
    xk_s[...] = _dot(oht, x_ref[0], ((0,), (0,))).astype(jnp.bfloat16)
    y = _ln_rows(xk_s[...].astype(jnp.float32), g1_ref[0], b1_ref[0]).astype(jnp.bfloat16)
    qkv_s[...] = (_dot(y, wq_s[...], ((1,), (0,))) + bq_ref[0]).astype(jnp.bfloat16)

    scale = 1.0 / np.sqrt(_DH)
    qkv = qkv_s[...]
    for h in range(_H):
        q = qkv[:, h * _DH:(h + 1) * _DH]
        k = qkv[:, _D + h * _DH:_D + (h + 1) * _DH]
        v = qkv[:, 2 * _D + h * _DH:2 * _D + (h + 1) * _DH]
        e = jnp.exp(_dot(q, k, ((1,), (1,))) * scale)
        att = (e / jnp.sum(e, axis=1, keepdims=True)).astype(jnp.bfloat16)
        o_s[:, h * _DH:(h + 1) * _DH] = _dot(att, v, ((1,), (0,))).astype(jnp.bfloat16)

    x2_ref[0] = (xk_s[...].astype(jnp.float32)
                 + _dot(o_s[...], wp_s[...], ((1,), (0,))) + bp_ref[0]
                 ).astype(jnp.bfloat16)


def _mlp_body(x2_ref, g_ref, w1_hbm, b1_ref, w2_hbm, b2_ref, g2_ref, bb2_ref,
              out_ref, w1_s, w2_s, sem1, sem2):
    b = pl.program_id(0)

    @pl.when(b == 0)
    def _():
        c1 = pltpu.make_async_copy(w1_hbm, w1_s, sem1)
        c2 = pltpu.make_async_copy(w2_hbm, w2_s, sem2)
        c1.start()
        c2.start()
        c1.wait()
        c2.wait()

    x2 = x2_ref[0].astype(jnp.float32)
    y2 = _ln_rows(x2, g2_ref[0], bb2_ref[0]).astype(jnp.bfloat16)
    hh = _DF // 2
    acc = None
    for u in range(2):
        h = jax.nn.gelu(_dot(y2, w1_s[:, u * hh:(u + 1) * hh], ((1,), (0,)))
                        + b1_ref[0, u * hh:(u + 1) * hh]).astype(jnp.bfloat16)
        c = _dot(h, w2_s[u * hh:(u + 1) * hh, :], ((1,), (0,)))
        acc = c if acc is None else acc + c
    out2 = (x2 + acc + b2_ref[0]).astype(jnp.bfloat16)

    g_row = g_ref[0]
    iota_j = jax.lax.broadcasted_iota(jnp.int32, (_NK, _N), 0)
    rt = (iota_j == g_row).astype(jnp.bfloat16)
    out_ref[0] = _dot(rt, out2, ((0,), (0,)))


def kernel(x, noise, ln1_g, ln1_b, ln2_g, ln2_b, w_qkv, b_qkv, w_proj, b_proj,
           w_fc1, b_fc1, w_fc2, b_fc2):
    keep_idx, g = _prep_indices(x, noise)
    keep3 = keep_idx.reshape(_B, 1, _NK)
    g3 = g.reshape(_B, 1, _N)
    r2 = lambda v: v.reshape(1, -1)
    bf = lambda v: v.astype(jnp.bfloat16)

    x2 = pl.pallas_call(
        _attn_body,
        grid=(_B,),
        in_specs=[
            pl.BlockSpec((1, _N, _D), lambda b: (b, 0, 0)),
            pl.BlockSpec((1, 1, _NK), lambda b: (b, 0, 0)),
            pl.BlockSpec(memory_space=pl.ANY),
            pl.BlockSpec((1, 3 * _D), lambda b: (0, 0)),
            pl.BlockSpec((1, _D), lambda b: (0, 0)),
            pl.BlockSpec((1, _D), lambda b: (0, 0)),
            pl.BlockSpec(memory_space=pl.ANY),
            pl.BlockSpec((1, _D), lambda b: (0, 0)),
        ],
        out_specs=pl.BlockSpec((1, _NK, _D), lambda b: (b, 0, 0)),
        out_shape=jax.ShapeDtypeStruct((_B, _NK, _D), jnp.bfloat16),
        scratch_shapes=[
            pltpu.VMEM((_D, 3 * _D), jnp.bfloat16),
            pltpu.VMEM((_D, _D), jnp.bfloat16),
            pltpu.VMEM((_NK, 3 * _D), jnp.bfloat16),
            pltpu.VMEM((_NK, _D), jnp.bfloat16),
            pltpu.VMEM((_NK, _D), jnp.bfloat16),
            pltpu.SemaphoreType.DMA,
            pltpu.SemaphoreType.DMA,
        ],
        compiler_params=pltpu.CompilerParams(
            dimension_semantics=("arbitrary",),
            vmem_limit_bytes=56 * 1024 * 1024,
        ),
        name="sito_gather_qkv_attn",
        interpret=_INTERPRET,
    )(bf(x), keep3, bf(w_qkv), r2(b_qkv), r2(ln1_g), r2(ln1_b), bf(w_proj), r2(b_proj))

    out = pl.pallas_call(
        _mlp_body,
        grid=(_B,),
        in_specs=[
            pl.BlockSpec((1, _NK, _D), lambda b: (b, 0, 0)),
            pl.BlockSpec((1, 1, _N), lambda b: (b, 0, 0)),
            pl.BlockSpec(memory_space=pl.ANY),
            pl.BlockSpec((1, _DF), lambda b: (0, 0)),
            pl.BlockSpec(memory_space=pl.ANY),
            pl.BlockSpec((1, _D), lambda b: (0, 0)),
            pl.BlockSpec((1, _D), lambda b: (0, 0)),
            pl.BlockSpec((1, _D), lambda b: (0, 0)),
        ],
        out_specs=pl.BlockSpec((1, _N, _D), lambda b: (b, 0, 0)),
        out_shape=jax.ShapeDtypeStruct((_B, _N, _D), jnp.float32),
        scratch_shapes=[
            pltpu.VMEM((_D, _DF), jnp.bfloat16),
            pltpu.VMEM((_DF, _D), jnp.bfloat16),
            pltpu.SemaphoreType.DMA,
            pltpu.SemaphoreType.DMA,
        ],
        compiler_params=pltpu.CompilerParams(
            dimension_semantics=("arbitrary",),
            vmem_limit_bytes=56 * 1024 * 1024,
        ),
        name="sito_mlp_recover",
        interpret=_INTERPRET,
    )(x2, g3, bf(w_fc1), r2(b_fc1), bf(w_fc2), r2(b_fc2), r2(ln2_g), r2(ln2_b))

    return out
